# Initial kernel scaffold; baseline (speedup 1.0000x reference)
import jax
import jax.numpy as jnp
from jax.experimental import pallas as pl


def kernel(pos, vel, box, box_feats, edge_index, box_src, box_dst, W0f, W0o, Wd0, bd0, Wc1, Wd1, bd1, Wc2, Wd2, bd2, Wc3, Wd3, bd3):
    raise NotImplementedError("write your pallas kernel here")


# speedup vs baseline: 4.4789x; 4.4789x over previous
import functools

import jax
import jax.numpy as jnp
from jax import lax
from jax.experimental import pallas as pl
from jax.experimental.pallas import tpu as pltpu
from jax.experimental.pallas import tpu_sc as plsc

NC = 2
NS = 16
LN = 16
CHUNK = 128

DT = 0.02
INV_H2 = float(1.0 / ((1.5 * 6 * 0.025 / 2.0) ** 2))


def _cdiv(a, b):
    return (a + b - 1) // b


def _row_spec(blk, w):
    return pl.BlockSpec((blk, w), lambda i: (i, 0))


def _full_spec(shape):
    return pl.BlockSpec(shape, lambda i: tuple(0 for _ in shape))


def _stage0_body(pos_ref, vel_ref, dtg_ref, wf_ref, bf_ref, wd_ref, bd_ref,
                 pos2_ref, yf_ref, d0_ref):
    vel = vel_ref[...]
    vel2 = vel + dtg_ref[...]
    pos2_ref[...] = pos_ref[...] + (0.5 * DT) * (vel2 + vel)
    yf_ref[...] = jnp.dot(vel2, wf_ref[...],
                          preferred_element_type=jnp.float32) + bf_ref[...]
    d0_ref[...] = jnp.dot(vel2, wd_ref[...],
                          preferred_element_type=jnp.float32) + bd_ref[...]


def _stage0(pos4, vel4, wfs, bf, wds, bd, blk=2000):
    n = pos4.shape[0]
    grid = n // blk
    dtg = jnp.array([[0.0, -9.81 * DT, 0.0, 0.0]], dtype=jnp.float32)
    return pl.pallas_call(
        _stage0_body,
        grid=(grid,),
        in_specs=[_row_spec(blk, 4), _row_spec(blk, 4), _full_spec((1, 4)),
                  _full_spec((4, 32)), _full_spec((1, 32)),
                  _full_spec((4, 32)), _full_spec((1, 32))],
        out_specs=[_row_spec(blk, 4), _row_spec(blk, 32), _row_spec(blk, 32)],
        out_shape=[jax.ShapeDtypeStruct((n, 4), jnp.float32),
                   jax.ShapeDtypeStruct((n, 32), jnp.float32),
                   jax.ShapeDtypeStruct((n, 32), jnp.float32)],
    )(pos4, vel4, dtg, wfs, bf, wds, bd)


def _boxmm_body(bf_ref, w_ref, y_ref):
    y_ref[...] = jnp.dot(bf_ref[...], w_ref[...],
                         preferred_element_type=jnp.float32)


def _boxmm(bf4, w0o4, blk=2000):
    nb = bf4.shape[0]
    return pl.pallas_call(
        _boxmm_body,
        grid=(nb // blk,),
        in_specs=[_row_spec(blk, 4), _full_spec((4, 32))],
        out_specs=_row_spec(blk, 32),
        out_shape=jax.ShapeDtypeStruct((nb, 32), jnp.float32),
    )(bf4, w0o4)


def _stage1_body(so_ref, sf_ref, d0_ref, wca_ref, wcb_ref, wd_ref, bd_ref,
                 ya_ref, yb_ref, d1_ref):
    h = jnp.concatenate(
        [so_ref[0] + so_ref[1], sf_ref[0] + sf_ref[1], d0_ref[...]], axis=-1)
    h = jnp.maximum(h, 0.0)
    ya_ref[...] = jnp.dot(h, wca_ref[...], preferred_element_type=jnp.float32)
    yb_ref[...] = jnp.dot(h, wcb_ref[...], preferred_element_type=jnp.float32)
    d1_ref[...] = jnp.dot(h, wd_ref[...],
                          preferred_element_type=jnp.float32) + bd_ref[...]


def _stage1(so, sf, d0, wca, wcb, wd, bd, blk=2000):
    n = d0.shape[0]
    sspec = pl.BlockSpec((2, blk, 32), lambda i: (0, i, 0))
    return pl.pallas_call(
        _stage1_body,
        grid=(n // blk,),
        in_specs=[sspec, sspec, _row_spec(blk, 32),
                  _full_spec((96, 32)), _full_spec((96, 32)),
                  _full_spec((96, 64)), _full_spec((1, 64))],
        out_specs=[_row_spec(blk, 32), _row_spec(blk, 32), _row_spec(blk, 64)],
        out_shape=[jax.ShapeDtypeStruct((n, 32), jnp.float32),
                   jax.ShapeDtypeStruct((n, 32), jnp.float32),
                   jax.ShapeDtypeStruct((n, 64), jnp.float32)],
    )(so, sf, d0, wca, wcb, wd, bd)


def _stage2_body(s_ref, d_ref, wca_ref, wcb_ref, wd_ref, bd_ref,
                 ya_ref, yb_ref, d2_ref):
    ans = jnp.concatenate([s_ref[0], s_ref[1]], axis=-1) + d_ref[...]
    h = jnp.maximum(ans, 0.0)
    ya_ref[...] = jnp.dot(h, wca_ref[...], preferred_element_type=jnp.float32)
    yb_ref[...] = jnp.dot(h, wcb_ref[...], preferred_element_type=jnp.float32)
    d2_ref[...] = jnp.dot(h, wd_ref[...],
                          preferred_element_type=jnp.float32) + bd_ref[...] + ans


def _stage2(s1, d1, wca, wcb, wd, bd, blk=2000):
    n = d1.shape[0]
    sspec = pl.BlockSpec((2, blk, 32), lambda i: (0, i, 0))
    return pl.pallas_call(
        _stage2_body,
        grid=(n // blk,),
        in_specs=[sspec, _row_spec(blk, 64),
                  _full_spec((64, 32)), _full_spec((64, 32)),
                  _full_spec((64, 64)), _full_spec((1, 64))],
        out_specs=[_row_spec(blk, 32), _row_spec(blk, 32), _row_spec(blk, 64)],
        out_shape=[jax.ShapeDtypeStruct((n, 32), jnp.float32),
                   jax.ShapeDtypeStruct((n, 32), jnp.float32),
                   jax.ShapeDtypeStruct((n, 64), jnp.float32)],
    )(s1, d1, wca, wcb, wd, bd)


def _stage3_body(s_ref, d_ref, wc_ref, wd_ref, bd_ref, y3_ref, d3_ref):
    ans = jnp.concatenate([s_ref[0], s_ref[1]], axis=-1) + d_ref[...]
    h = jnp.maximum(ans, 0.0)
    y3_ref[...] = jnp.dot(h, wc_ref[...], preferred_element_type=jnp.float32)
    d3_ref[...] = jnp.dot(h, wd_ref[...],
                          preferred_element_type=jnp.float32) + bd_ref[...]


def _stage3(s2, d2, wc3p, wd3p, bd3p, blk=2000):
    n = d2.shape[0]
    sspec = pl.BlockSpec((2, blk, 32), lambda i: (0, i, 0))
    return pl.pallas_call(
        _stage3_body,
        grid=(n // blk,),
        in_specs=[sspec, _row_spec(blk, 64),
                  _full_spec((64, 16)), _full_spec((64, 16)),
                  _full_spec((1, 16))],
        out_specs=[_row_spec(blk, 16), _row_spec(blk, 16)],
        out_shape=[jax.ShapeDtypeStruct((n, 16), jnp.float32),
                   jax.ShapeDtypeStruct((n, 16), jnp.float32)],
    )(s2, d2, wc3p, wd3p, bd3p)


def _sc_mesh():
    return plsc.VectorSubcoreMesh(core_axis_name="c", subcore_axis_name="s")


def _win_kernel(e_real, ep):
    per_tile = ep // (NC * NS)
    nchunk = per_tile // CHUNK

    @functools.partial(
        pl.kernel,
        out_type=jax.ShapeDtypeStruct((ep,), jnp.float32),
        mesh=_sc_mesh(),
        compiler_params=pltpu.CompilerParams(use_tc_tiling_on_sc=False),
        scratch_types=[
            pltpu.VMEM((CHUNK,), jnp.int32),
            pltpu.VMEM((CHUNK,), jnp.int32),
            [pltpu.VMEM((CHUNK,), jnp.float32) for _ in range(6)],
            pltpu.VMEM((CHUNK,), jnp.float32),
            [pltpu.SemaphoreType.DMA for _ in range(6)],
        ],
    )
    def k(sx_hbm, sy_hbm, sz_hbm, dx_hbm, dy_hbm, dz_hbm,
          src_hbm, dst_hbm, win_hbm,
          sidx_v, didx_v, crd_v, wout_v, sems):
        c = lax.axis_index("c")
        s = lax.axis_index("s")
        base0 = (c * NS + s) * per_tile
        iota = lax.iota(jnp.int32, LN)

        def chunk_body(j, _):
            base = base0 + j * CHUNK
            pltpu.sync_copy(src_hbm.at[pl.ds(base, CHUNK)], sidx_v)
            pltpu.sync_copy(dst_hbm.at[pl.ds(base, CHUNK)], didx_v)
            cps = []
            for t, (tbl, idx) in enumerate(
                    [(sx_hbm, sidx_v), (sy_hbm, sidx_v), (sz_hbm, sidx_v),
                     (dx_hbm, didx_v), (dy_hbm, didx_v), (dz_hbm, didx_v)]):
                cps.append(pltpu.async_copy(tbl.at[idx], crd_v[t], sems[t]))
            for cp in cps:
                cp.wait()

            def sub(i, _):
                sl = pl.ds(i * LN, LN)
                dx = crd_v[3][sl] - crd_v[0][sl]
                dy = crd_v[4][sl] - crd_v[1][sl]
                dz = crd_v[5][sl] - crd_v[2][sl]
                r2 = (dx * dx + dy * dy + dz * dz) * INV_H2
                t1 = 1.0 - r2
                w = t1 * t1 * t1
                w = jnp.minimum(jnp.maximum(w, 0.0), 1.0)
                g = base + i * LN + iota
                w = jnp.where(g < e_real, w, 0.0)
                wout_v[sl] = w
                return 0

            lax.fori_loop(0, CHUNK // LN, sub, 0)
            pltpu.sync_copy(wout_v, win_hbm.at[pl.ds(base, CHUNK)])
            return 0

        lax.fori_loop(0, nchunk, chunk_body, 0)

    return k


def _seg_kernel(n, ep, width, colsplit):
    per_sc = ep if colsplit else ep // NC
    per_tile = per_sc // NS
    nchunk = per_tile // CHUNK
    rpt = ((_cdiv(n, NS) + 7) // 8) * 8
    assert per_tile % CHUNK == 0 and width % LN == 0 and (n - rpt) % 8 == 0

    @functools.partial(
        pl.kernel,
        out_type=jax.ShapeDtypeStruct((NC, n, width), jnp.float32),
        mesh=_sc_mesh(),
        compiler_params=pltpu.CompilerParams(use_tc_tiling_on_sc=False),
        scratch_types=[
            pltpu.VMEM((CHUNK,), jnp.int32),
            pltpu.VMEM((CHUNK,), jnp.int32),
            pltpu.VMEM((CHUNK,), jnp.float32),
            pltpu.VMEM((CHUNK, width), jnp.float32),
            pltpu.VMEM_SHARED((n, width), jnp.float32),
            pltpu.SemaphoreType.DMA,
        ],
    )
    def k(tbl_hbm, src_hbm, dst_hbm, win_hbm, zeros_hbm, out_hbm,
          sidx_v, didx_v, win_v, rows_v, accum_sh, semg):
        c = lax.axis_index("c")
        s = lax.axis_index("s")
        if colsplit:
            ebase0 = s * per_tile
            sbase0 = c * ep + s * per_tile
        else:
            ebase0 = (c * per_sc + s * per_tile)
            sbase0 = ebase0
        row0 = pl.multiple_of(jnp.minimum(s * rpt, n - rpt), 8)

        pltpu.sync_copy(zeros_hbm.at[pl.ds(row0, rpt)],
                        accum_sh.at[pl.ds(row0, rpt)])
        plsc.subcore_barrier()

        def chunk_body(j, _):
            eb = ebase0 + j * CHUNK
            sb = sbase0 + j * CHUNK
            pltpu.sync_copy(src_hbm.at[pl.ds(sb, CHUNK)], sidx_v)
            pltpu.sync_copy(dst_hbm.at[pl.ds(eb, CHUNK)], didx_v)
            pltpu.sync_copy(win_hbm.at[pl.ds(eb, CHUNK)], win_v)
            pltpu.async_copy(tbl_hbm.at[sidx_v], rows_v, semg).wait()

            def sgrp(g, _):
                win16 = win_v[pl.ds(g * LN, LN)]
                for lane in range(LN):
                    w = win16[lane]
                    i = g * LN + lane
                    for colb in range(width // LN):
                        sl = pl.ds(colb * LN, LN)
                        rows_v[i, sl] = rows_v[i, sl] * w
                return 0
            lax.fori_loop(0, CHUNK // LN, sgrp, 0)

            pltpu.sync_copy(rows_v, accum_sh.at[didx_v], add=True)
            return 0

        lax.fori_loop(0, nchunk, chunk_body, 0)
        plsc.subcore_barrier()
        pltpu.sync_copy(accum_sh.at[pl.ds(row0, rpt)],
                        out_hbm.at[c, pl.ds(row0, rpt)])

    return k


def kernel(pos, vel, box, box_feats, edge_index, box_src, box_dst,
           W0f, W0o, Wd0, bd0, Wc1, Wd1, bd1, Wc2, Wd2, bd2, Wc3, Wd3, bd3):
    n = pos.shape[0]
    nb = box.shape[0]
    e = edge_index.shape[1]
    eb = box_src.shape[0]
    grain = NC * NS * CHUNK
    ep = _cdiv(e, grain) * grain
    ebp = _cdiv(eb, grain) * grain

    pad1 = ((0, 0), (0, 1))
    pos4 = jnp.pad(pos, pad1)
    vel4 = jnp.pad(vel, pad1)
    bf4 = jnp.pad(box_feats, pad1)
    w0o4 = jnp.pad(W0o, ((0, 1), (0, 0)))

    wfs = jnp.pad(W0f[1:4], ((0, 1), (0, 0)))
    bf0 = W0f[0:1]
    wds = jnp.pad(Wd0[1:4], ((0, 1), (0, 0)))
    bd0r = Wd0[0:1] + bd0[None, :]

    wc1a, wc1b = Wc1[:, :32], Wc1[:, 32:]
    wc2a, wc2b = Wc2[:, :32], Wc2[:, 32:]
    wc3p = jnp.pad(Wc3, ((0, 0), (0, 13)))
    wd3p = jnp.pad(Wd3, ((0, 0), (0, 13)))
    bd3p = jnp.pad(bd3, (0, 13))[None, :]

    srcp = jnp.pad(edge_index[0], (0, ep - e))
    dstp = jnp.pad(edge_index[1], (0, ep - e))
    bsrcp = jnp.pad(box_src, (0, ebp - eb))
    bdstp = jnp.pad(box_dst, (0, ebp - eb))
    src_off = jnp.concatenate([srcp, srcp + n])

    zeros32 = jnp.zeros((n, 32), jnp.float32)
    zeros16 = jnp.zeros((n, 16), jnp.float32)

    pos2p, yf, d0 = _stage0(pos4, vel4, wfs, bf0, wds, bd0r)
    ybox = _boxmm(bf4, w0o4)

    p2x, p2y, p2z = pos2p[:, 0], pos2p[:, 1], pos2p[:, 2]
    bx, by, bz = box[:, 0], box[:, 1], box[:, 2]
    winf = _win_kernel(e, ep)(p2x, p2y, p2z, p2x, p2y, p2z, srcp, dstp)
    winb = _win_kernel(eb, ebp)(bx, by, bz, p2x, p2y, p2z, bsrcp, bdstp)

    sf = _seg_kernel(n, ep, 32, False)(yf, srcp, dstp, winf, zeros32)
    so = _seg_kernel(n, ebp, 32, False)(ybox, bsrcp, bdstp, winb, zeros32)

    y1a, y1b, d1 = _stage1(so, sf, d0, wc1a, wc1b, Wd1, bd1[None, :])
    y1t = jnp.concatenate([y1a, y1b], axis=0)
    s1 = _seg_kernel(n, ep, 32, True)(y1t, src_off, dstp, winf, zeros32)

    y2a, y2b, d2 = _stage2(s1, d1, wc2a, wc2b, Wd2, bd2[None, :])
    y2t = jnp.concatenate([y2a, y2b], axis=0)
    s2 = _seg_kernel(n, ep, 32, True)(y2t, src_off, dstp, winf, zeros32)

    y3, d3 = _stage3(s2, d2, wc3p, wd3p, bd3p)
    s3 = _seg_kernel(n, ep, 16, False)(y3, srcp, dstp, winf, zeros16)

    out = (s3[0] + s3[1] + d3)[:, :3] * (1.0 / 128.0)
    return out


# speedup vs baseline: 6.7203x; 1.5004x over previous
import functools

import jax
import jax.numpy as jnp
from jax import lax
from jax.experimental import pallas as pl
from jax.experimental.pallas import tpu as pltpu
from jax.experimental.pallas import tpu_sc as plsc

NC = 2
NS = 16
LN = 16
CHUNK = 128

DT = 0.02
INV_H2 = float(1.0 / ((1.5 * 6 * 0.025 / 2.0) ** 2))


def _cdiv(a, b):
    return (a + b - 1) // b


def _row_spec(blk, w):
    return pl.BlockSpec((blk, w), lambda i: (i, 0))


def _full_spec(shape):
    return pl.BlockSpec(shape, lambda i: tuple(0 for _ in shape))


def _stage0_body(pos_ref, vel_ref, dtg_ref, wf_ref, bf_ref, wd_ref, bd_ref,
                 pos2_ref, yf_ref, d0_ref):
    vel = vel_ref[...]
    vel2 = vel + dtg_ref[...]
    pos2_ref[...] = pos_ref[...] + (0.5 * DT) * (vel2 + vel)
    yf_ref[...] = jnp.dot(vel2, wf_ref[...],
                          preferred_element_type=jnp.float32) + bf_ref[...]
    d0_ref[...] = jnp.dot(vel2, wd_ref[...],
                          preferred_element_type=jnp.float32) + bd_ref[...]


def _stage0(pos4, vel4, wfs, bf, wds, bd, blk=2000):
    n = pos4.shape[0]
    grid = n // blk
    dtg = jnp.array([[0.0, -9.81 * DT, 0.0, 0.0]], dtype=jnp.float32)
    return pl.pallas_call(
        _stage0_body,
        grid=(grid,),
        in_specs=[_row_spec(blk, 4), _row_spec(blk, 4), _full_spec((1, 4)),
                  _full_spec((4, 32)), _full_spec((1, 32)),
                  _full_spec((4, 32)), _full_spec((1, 32))],
        out_specs=[_row_spec(blk, 4), _row_spec(blk, 32), _row_spec(blk, 32)],
        out_shape=[jax.ShapeDtypeStruct((n, 4), jnp.float32),
                   jax.ShapeDtypeStruct((n, 32), jnp.float32),
                   jax.ShapeDtypeStruct((n, 32), jnp.float32)],
    )(pos4, vel4, dtg, wfs, bf, wds, bd)


def _boxmm_body(bf_ref, w_ref, y_ref):
    y_ref[...] = jnp.dot(bf_ref[...], w_ref[...],
                         preferred_element_type=jnp.float32)


def _boxmm(bf4, w0o4, blk=2000):
    nb = bf4.shape[0]
    return pl.pallas_call(
        _boxmm_body,
        grid=(nb // blk,),
        in_specs=[_row_spec(blk, 4), _full_spec((4, 32))],
        out_specs=_row_spec(blk, 32),
        out_shape=jax.ShapeDtypeStruct((nb, 32), jnp.float32),
    )(bf4, w0o4)


def _stage1_body(so_ref, sf_ref, d0_ref, wca_ref, wcb_ref, wd_ref, bd_ref,
                 ya_ref, yb_ref, d1_ref):
    h = jnp.concatenate(
        [so_ref[0] + so_ref[1], sf_ref[0] + sf_ref[1], d0_ref[...]], axis=-1)
    h = jnp.maximum(h, 0.0)
    ya_ref[...] = jnp.dot(h, wca_ref[...], preferred_element_type=jnp.float32)
    yb_ref[...] = jnp.dot(h, wcb_ref[...], preferred_element_type=jnp.float32)
    d1_ref[...] = jnp.dot(h, wd_ref[...],
                          preferred_element_type=jnp.float32) + bd_ref[...]


def _stage1(so, sf, d0, wca, wcb, wd, bd, blk=2000):
    n = d0.shape[0]
    sspec = pl.BlockSpec((2, blk, 32), lambda i: (0, i, 0))
    return pl.pallas_call(
        _stage1_body,
        grid=(n // blk,),
        in_specs=[sspec, sspec, _row_spec(blk, 32),
                  _full_spec((96, 32)), _full_spec((96, 32)),
                  _full_spec((96, 64)), _full_spec((1, 64))],
        out_specs=[_row_spec(blk, 32), _row_spec(blk, 32), _row_spec(blk, 64)],
        out_shape=[jax.ShapeDtypeStruct((n, 32), jnp.float32),
                   jax.ShapeDtypeStruct((n, 32), jnp.float32),
                   jax.ShapeDtypeStruct((n, 64), jnp.float32)],
    )(so, sf, d0, wca, wcb, wd, bd)


def _stage2_body(s_ref, d_ref, wca_ref, wcb_ref, wd_ref, bd_ref,
                 ya_ref, yb_ref, d2_ref):
    ans = jnp.concatenate([s_ref[0], s_ref[1]], axis=-1) + d_ref[...]
    h = jnp.maximum(ans, 0.0)
    ya_ref[...] = jnp.dot(h, wca_ref[...], preferred_element_type=jnp.float32)
    yb_ref[...] = jnp.dot(h, wcb_ref[...], preferred_element_type=jnp.float32)
    d2_ref[...] = jnp.dot(h, wd_ref[...],
                          preferred_element_type=jnp.float32) + bd_ref[...] + ans


def _stage2(s1, d1, wca, wcb, wd, bd, blk=2000):
    n = d1.shape[0]
    sspec = pl.BlockSpec((2, blk, 32), lambda i: (0, i, 0))
    return pl.pallas_call(
        _stage2_body,
        grid=(n // blk,),
        in_specs=[sspec, _row_spec(blk, 64),
                  _full_spec((64, 32)), _full_spec((64, 32)),
                  _full_spec((64, 64)), _full_spec((1, 64))],
        out_specs=[_row_spec(blk, 32), _row_spec(blk, 32), _row_spec(blk, 64)],
        out_shape=[jax.ShapeDtypeStruct((n, 32), jnp.float32),
                   jax.ShapeDtypeStruct((n, 32), jnp.float32),
                   jax.ShapeDtypeStruct((n, 64), jnp.float32)],
    )(s1, d1, wca, wcb, wd, bd)


def _stage3_body(s_ref, d_ref, wc_ref, wd_ref, bd_ref, y3_ref, d3_ref):
    ans = jnp.concatenate([s_ref[0], s_ref[1]], axis=-1) + d_ref[...]
    h = jnp.maximum(ans, 0.0)
    y3_ref[...] = jnp.dot(h, wc_ref[...], preferred_element_type=jnp.float32)
    d3_ref[...] = jnp.dot(h, wd_ref[...],
                          preferred_element_type=jnp.float32) + bd_ref[...]


def _stage3(s2, d2, wc3p, wd3p, bd3p, blk=2000):
    n = d2.shape[0]
    sspec = pl.BlockSpec((2, blk, 32), lambda i: (0, i, 0))
    return pl.pallas_call(
        _stage3_body,
        grid=(n // blk,),
        in_specs=[sspec, _row_spec(blk, 64),
                  _full_spec((64, 16)), _full_spec((64, 16)),
                  _full_spec((1, 16))],
        out_specs=[_row_spec(blk, 16), _row_spec(blk, 16)],
        out_shape=[jax.ShapeDtypeStruct((n, 16), jnp.float32),
                   jax.ShapeDtypeStruct((n, 16), jnp.float32)],
    )(s2, d2, wc3p, wd3p, bd3p)


def _sc_mesh():
    return plsc.VectorSubcoreMesh(core_axis_name="c", subcore_axis_name="s")


def _win_kernel(e_real, ep):
    per_tile = ep // (NC * NS)
    nchunk = per_tile // CHUNK

    @functools.partial(
        pl.kernel,
        out_type=jax.ShapeDtypeStruct((ep // CHUNK, CHUNK), jnp.float32),
        mesh=_sc_mesh(),
        compiler_params=pltpu.CompilerParams(use_tc_tiling_on_sc=False),
        scratch_types=[
            pltpu.VMEM((CHUNK,), jnp.int32),
            pltpu.VMEM((CHUNK,), jnp.int32),
            [pltpu.VMEM((CHUNK,), jnp.float32) for _ in range(6)],
            pltpu.VMEM((CHUNK,), jnp.float32),
            [pltpu.SemaphoreType.DMA for _ in range(6)],
        ],
    )
    def k(sx_hbm, sy_hbm, sz_hbm, dx_hbm, dy_hbm, dz_hbm,
          src_hbm, dst_hbm, win_hbm,
          sidx_v, didx_v, crd_v, wout_v, sems):
        c = lax.axis_index("c")
        s = lax.axis_index("s")
        brow = (c * NS + s) * nchunk
        iota = lax.iota(jnp.int32, LN)

        def chunk_body(j, _):
            row = brow + j
            pltpu.sync_copy(src_hbm.at[row], sidx_v)
            pltpu.sync_copy(dst_hbm.at[row], didx_v)
            cps = []
            for t, (tbl, idx) in enumerate(
                    [(sx_hbm, sidx_v), (sy_hbm, sidx_v), (sz_hbm, sidx_v),
                     (dx_hbm, didx_v), (dy_hbm, didx_v), (dz_hbm, didx_v)]):
                cps.append(pltpu.async_copy(tbl.at[idx], crd_v[t], sems[t]))
            for cp in cps:
                cp.wait()

            def sub(i, _):
                sl = pl.ds(i * LN, LN)
                dx = crd_v[3][sl] - crd_v[0][sl]
                dy = crd_v[4][sl] - crd_v[1][sl]
                dz = crd_v[5][sl] - crd_v[2][sl]
                r2 = (dx * dx + dy * dy + dz * dz) * INV_H2
                t1 = 1.0 - r2
                w = t1 * t1 * t1
                w = jnp.minimum(jnp.maximum(w, 0.0), 1.0)
                g = row * CHUNK + i * LN + iota
                w = jnp.where(g < e_real, w, 0.0)
                wout_v[sl] = w
                return 0

            lax.fori_loop(0, CHUNK // LN, sub, 0)
            pltpu.sync_copy(wout_v, win_hbm.at[row])
            return 0

        lax.fori_loop(0, nchunk, chunk_body, 0)

    return k


NSUP = 4


def _seg_kernel(n, ep, width, colsplit):
    per_sc = ep if colsplit else ep // NC
    per_tile = per_sc // NS
    nchunk = per_tile // CHUNK
    nsup = nchunk // NSUP
    rpt = ((_cdiv(n, NS) + 7) // 8) * 8
    assert per_tile % (CHUNK * NSUP) == 0 and nsup % 2 == 0
    assert width % LN == 0 and (n - rpt) % 8 == 0

    @functools.partial(
        pl.kernel,
        out_type=jax.ShapeDtypeStruct((NC, n, width), jnp.float32),
        mesh=_sc_mesh(),
        compiler_params=pltpu.CompilerParams(use_tc_tiling_on_sc=False),
        scratch_types=[
            pltpu.VMEM((2, NSUP, CHUNK), jnp.int32),
            pltpu.VMEM((2, NSUP, CHUNK), jnp.int32),
            pltpu.VMEM((2, NSUP, CHUNK), jnp.float32),
            pltpu.VMEM((NSUP * CHUNK, width), jnp.float32),
            pltpu.VMEM_SHARED((n, width), jnp.float32),
            pltpu.SemaphoreType.DMA,
            pltpu.SemaphoreType.DMA,
            pltpu.SemaphoreType.DMA,
        ],
    )
    def k(tbl_hbm, src_hbm, dst_hbm, win_hbm, zeros_hbm, out_hbm,
          sidx_v, didx_v, win_v, rows_v, accum_sh, semi, semg, semsc):
        c = lax.axis_index("c")
        s = lax.axis_index("s")
        if colsplit:
            erow0 = s * nchunk
            srow0 = c * (ep // CHUNK) + s * nchunk
        else:
            erow0 = c * (per_sc // CHUNK) + s * nchunk
            srow0 = erow0
        row0 = pl.multiple_of(jnp.minimum(s * rpt, n - rpt), 8)

        pltpu.sync_copy(zeros_hbm.at[pl.ds(row0, rpt)],
                        accum_sh.at[pl.ds(row0, rpt)])
        plsc.subcore_barrier()

        def idx_copies(b, p):
            r = b * NSUP
            return [
                pltpu.make_async_copy(
                    src_hbm.at[pl.ds(srow0 + r, NSUP)], sidx_v.at[p], semi),
                pltpu.make_async_copy(
                    dst_hbm.at[pl.ds(erow0 + r, NSUP)], didx_v.at[p], semi),
                pltpu.make_async_copy(
                    win_hbm.at[pl.ds(erow0 + r, NSUP)], win_v.at[p], semi),
            ]

        def issue_idx(b, p):
            for cp in idx_copies(b, p):
                cp.start()

        def process(b, p):
            gs = []
            for q in range(NSUP):
                gs.append(pltpu.async_copy(
                    tbl_hbm.at[sidx_v.at[p, q]],
                    rows_v.at[pl.ds(q * CHUNK, CHUNK)], semg))
            ss = []
            for q in range(NSUP):
                gs[q].wait()

                def sgrp(g, _, q=q):
                    win16 = win_v[p, q, pl.ds(g * LN, LN)]
                    for lane in range(LN):
                        w = win16[lane]
                        i = q * CHUNK + g * LN + lane
                        for colb in range(width // LN):
                            sl = pl.ds(colb * LN, LN)
                            rows_v[i, sl] = rows_v[i, sl] * w
                    return 0
                lax.fori_loop(0, CHUNK // LN, sgrp, 0)
                ss.append(pltpu.async_copy(
                    rows_v.at[pl.ds(q * CHUNK, CHUNK)],
                    accum_sh.at[didx_v.at[p, q]], semsc, add=True))
            for cp in ss:
                cp.wait()

        issue_idx(0, 0)

        def pair(i, _):
            b0 = 2 * i
            b1 = 2 * i + 1
            for cp in idx_copies(b0, 0):
                cp.wait()
            issue_idx(b1, 1)
            process(b0, 0)
            for cp in idx_copies(b1, 1):
                cp.wait()

            @pl.when(b1 + 1 < nsup)
            def _():
                issue_idx(b1 + 1, 0)
            process(b1, 1)
            return 0

        lax.fori_loop(0, nsup // 2, pair, 0)
        plsc.subcore_barrier()
        pltpu.sync_copy(accum_sh.at[pl.ds(row0, rpt)],
                        out_hbm.at[c, pl.ds(row0, rpt)])

    return k


def kernel(pos, vel, box, box_feats, edge_index, box_src, box_dst,
           W0f, W0o, Wd0, bd0, Wc1, Wd1, bd1, Wc2, Wd2, bd2, Wc3, Wd3, bd3):
    n = pos.shape[0]
    nb = box.shape[0]
    e = edge_index.shape[1]
    eb = box_src.shape[0]
    grain = NC * NS * CHUNK * 2 * NSUP
    ep = _cdiv(e, grain) * grain
    ebp = _cdiv(eb, grain) * grain

    pad1 = ((0, 0), (0, 1))
    pos4 = jnp.pad(pos, pad1)
    vel4 = jnp.pad(vel, pad1)
    bf4 = jnp.pad(box_feats, pad1)
    w0o4 = jnp.pad(W0o, ((0, 1), (0, 0)))

    wfs = jnp.pad(W0f[1:4], ((0, 1), (0, 0)))
    bf0 = W0f[0:1]
    wds = jnp.pad(Wd0[1:4], ((0, 1), (0, 0)))
    bd0r = Wd0[0:1] + bd0[None, :]

    wc1a, wc1b = Wc1[:, :32], Wc1[:, 32:]
    wc2a, wc2b = Wc2[:, :32], Wc2[:, 32:]
    wc3p = jnp.pad(Wc3, ((0, 0), (0, 13)))
    wd3p = jnp.pad(Wd3, ((0, 0), (0, 13)))
    bd3p = jnp.pad(bd3, (0, 13))[None, :]

    srcp = jnp.pad(edge_index[0], (0, ep - e))
    dstp = jnp.pad(edge_index[1], (0, ep - e))
    bsrcp = jnp.pad(box_src, (0, ebp - eb))
    bdstp = jnp.pad(box_dst, (0, ebp - eb))
    src_off = jnp.concatenate([srcp, srcp + n])
    src2 = srcp.reshape(-1, CHUNK)
    dst2 = dstp.reshape(-1, CHUNK)
    bsrc2 = bsrcp.reshape(-1, CHUNK)
    bdst2 = bdstp.reshape(-1, CHUNK)
    srco2 = src_off.reshape(-1, CHUNK)

    zeros32 = jnp.zeros((n, 32), jnp.float32)
    zeros16 = jnp.zeros((n, 16), jnp.float32)

    pos2p, yf, d0 = _stage0(pos4, vel4, wfs, bf0, wds, bd0r)
    ybox = _boxmm(bf4, w0o4)

    p2x, p2y, p2z = pos2p[:, 0], pos2p[:, 1], pos2p[:, 2]
    bx, by, bz = box[:, 0], box[:, 1], box[:, 2]
    winf = _win_kernel(e, ep)(p2x, p2y, p2z, p2x, p2y, p2z, src2, dst2)
    winb = _win_kernel(eb, ebp)(bx, by, bz, p2x, p2y, p2z, bsrc2, bdst2)

    sf = _seg_kernel(n, ep, 32, False)(yf, src2, dst2, winf, zeros32)
    so = _seg_kernel(n, ebp, 32, False)(ybox, bsrc2, bdst2, winb, zeros32)

    y1a, y1b, d1 = _stage1(so, sf, d0, wc1a, wc1b, Wd1, bd1[None, :])
    y1t = jnp.concatenate([y1a, y1b], axis=0)
    s1 = _seg_kernel(n, ep, 32, True)(y1t, srco2, dst2, winf, zeros32)

    y2a, y2b, d2 = _stage2(s1, d1, wc2a, wc2b, Wd2, bd2[None, :])
    y2t = jnp.concatenate([y2a, y2b], axis=0)
    s2 = _seg_kernel(n, ep, 32, True)(y2t, srco2, dst2, winf, zeros32)

    y3, d3 = _stage3(s2, d2, wc3p, wd3p, bd3p)
    s3 = _seg_kernel(n, ep, 16, False)(y3, src2, dst2, winf, zeros16)

    out = (s3[0] + s3[1] + d3)[:, :3] * (1.0 / 128.0)
    return out


# speedup vs baseline: 7.2666x; 1.0813x over previous
import functools

import jax
import jax.numpy as jnp
from jax import lax
from jax.experimental import pallas as pl
from jax.experimental.pallas import tpu as pltpu
from jax.experimental.pallas import tpu_sc as plsc

NC = 2
NS = 16
LN = 16
CHUNK = 128

DT = 0.02
INV_H2 = float(1.0 / ((1.5 * 6 * 0.025 / 2.0) ** 2))


def _cdiv(a, b):
    return (a + b - 1) // b


def _row_spec(blk, w):
    return pl.BlockSpec((blk, w), lambda i: (i, 0))


def _full_spec(shape):
    return pl.BlockSpec(shape, lambda i: tuple(0 for _ in shape))


def _stage0_body(pos_ref, vel_ref, dtg_ref, wf_ref, bf_ref, wd_ref, bd_ref,
                 pos2_ref, yf_ref, d0_ref):
    vel = vel_ref[...]
    vel2 = vel + dtg_ref[...]
    pos2_ref[...] = pos_ref[...] + (0.5 * DT) * (vel2 + vel)
    yf_ref[...] = jnp.dot(vel2, wf_ref[...],
                          preferred_element_type=jnp.float32) + bf_ref[...]
    d0_ref[...] = jnp.dot(vel2, wd_ref[...],
                          preferred_element_type=jnp.float32) + bd_ref[...]


def _stage0(pos4, vel4, wfs, bf, wds, bd, blk=2000):
    n = pos4.shape[0]
    grid = n // blk
    dtg = jnp.array([[0.0, -9.81 * DT, 0.0, 0.0]], dtype=jnp.float32)
    return pl.pallas_call(
        _stage0_body,
        grid=(grid,),
        in_specs=[_row_spec(blk, 4), _row_spec(blk, 4), _full_spec((1, 4)),
                  _full_spec((4, 32)), _full_spec((1, 32)),
                  _full_spec((4, 32)), _full_spec((1, 32))],
        out_specs=[_row_spec(blk, 4), _row_spec(blk, 32), _row_spec(blk, 32)],
        out_shape=[jax.ShapeDtypeStruct((n, 4), jnp.float32),
                   jax.ShapeDtypeStruct((n, 32), jnp.float32),
                   jax.ShapeDtypeStruct((n, 32), jnp.float32)],
    )(pos4, vel4, dtg, wfs, bf, wds, bd)


def _boxmm_body(bf_ref, w_ref, y_ref):
    y_ref[...] = jnp.dot(bf_ref[...], w_ref[...],
                         preferred_element_type=jnp.float32)


def _boxmm(bf4, w0o4, blk=2000):
    nb = bf4.shape[0]
    return pl.pallas_call(
        _boxmm_body,
        grid=(nb // blk,),
        in_specs=[_row_spec(blk, 4), _full_spec((4, 32))],
        out_specs=_row_spec(blk, 32),
        out_shape=jax.ShapeDtypeStruct((nb, 32), jnp.float32),
    )(bf4, w0o4)


def _stage1_body(so_ref, sf_ref, d0_ref, wca_ref, wcb_ref, wd_ref, bd_ref,
                 ya_ref, yb_ref, d1_ref):
    h = jnp.concatenate(
        [so_ref[0] + so_ref[1], sf_ref[0] + sf_ref[1], d0_ref[...]], axis=-1)
    h = jnp.maximum(h, 0.0)
    ya_ref[...] = jnp.dot(h, wca_ref[...], preferred_element_type=jnp.float32)
    yb_ref[...] = jnp.dot(h, wcb_ref[...], preferred_element_type=jnp.float32)
    d1_ref[...] = jnp.dot(h, wd_ref[...],
                          preferred_element_type=jnp.float32) + bd_ref[...]


def _stage1(so, sf, d0, wca, wcb, wd, bd, blk=2000):
    n = d0.shape[0]
    sspec = pl.BlockSpec((2, blk, 32), lambda i: (0, i, 0))
    return pl.pallas_call(
        _stage1_body,
        grid=(n // blk,),
        in_specs=[sspec, sspec, _row_spec(blk, 32),
                  _full_spec((96, 32)), _full_spec((96, 32)),
                  _full_spec((96, 64)), _full_spec((1, 64))],
        out_specs=[_row_spec(blk, 32), _row_spec(blk, 32), _row_spec(blk, 64)],
        out_shape=[jax.ShapeDtypeStruct((n, 32), jnp.float32),
                   jax.ShapeDtypeStruct((n, 32), jnp.float32),
                   jax.ShapeDtypeStruct((n, 64), jnp.float32)],
    )(so, sf, d0, wca, wcb, wd, bd)


def _stage2_body(s_ref, d_ref, wca_ref, wcb_ref, wd_ref, bd_ref,
                 ya_ref, yb_ref, d2_ref):
    ans = jnp.concatenate([s_ref[0], s_ref[1]], axis=-1) + d_ref[...]
    h = jnp.maximum(ans, 0.0)
    ya_ref[...] = jnp.dot(h, wca_ref[...], preferred_element_type=jnp.float32)
    yb_ref[...] = jnp.dot(h, wcb_ref[...], preferred_element_type=jnp.float32)
    d2_ref[...] = jnp.dot(h, wd_ref[...],
                          preferred_element_type=jnp.float32) + bd_ref[...] + ans


def _stage2(s1, d1, wca, wcb, wd, bd, blk=2000):
    n = d1.shape[0]
    sspec = pl.BlockSpec((2, blk, 32), lambda i: (0, i, 0))
    return pl.pallas_call(
        _stage2_body,
        grid=(n // blk,),
        in_specs=[sspec, _row_spec(blk, 64),
                  _full_spec((64, 32)), _full_spec((64, 32)),
                  _full_spec((64, 64)), _full_spec((1, 64))],
        out_specs=[_row_spec(blk, 32), _row_spec(blk, 32), _row_spec(blk, 64)],
        out_shape=[jax.ShapeDtypeStruct((n, 32), jnp.float32),
                   jax.ShapeDtypeStruct((n, 32), jnp.float32),
                   jax.ShapeDtypeStruct((n, 64), jnp.float32)],
    )(s1, d1, wca, wcb, wd, bd)


def _stage3_body(s_ref, d_ref, wc_ref, wd_ref, bd_ref, y3_ref, d3_ref):
    ans = jnp.concatenate([s_ref[0], s_ref[1]], axis=-1) + d_ref[...]
    h = jnp.maximum(ans, 0.0)
    y3_ref[...] = jnp.dot(h, wc_ref[...], preferred_element_type=jnp.float32)
    d3_ref[...] = jnp.dot(h, wd_ref[...],
                          preferred_element_type=jnp.float32) + bd_ref[...]


def _stage3(s2, d2, wc3p, wd3p, bd3p, blk=2000):
    n = d2.shape[0]
    sspec = pl.BlockSpec((2, blk, 32), lambda i: (0, i, 0))
    return pl.pallas_call(
        _stage3_body,
        grid=(n // blk,),
        in_specs=[sspec, _row_spec(blk, 64),
                  _full_spec((64, 16)), _full_spec((64, 16)),
                  _full_spec((1, 16))],
        out_specs=[_row_spec(blk, 16), _row_spec(blk, 16)],
        out_shape=[jax.ShapeDtypeStruct((n, 16), jnp.float32),
                   jax.ShapeDtypeStruct((n, 16), jnp.float32)],
    )(s2, d2, wc3p, wd3p, bd3p)


def _sc_mesh():
    return plsc.VectorSubcoreMesh(core_axis_name="c", subcore_axis_name="s")


def _win_kernel(e_real, ep):
    per_tile = ep // (NC * NS)
    nchunk = per_tile // CHUNK
    nsw = NSUP
    nsup = nchunk // nsw
    assert nchunk % (2 * nsw) == 0

    @functools.partial(
        pl.kernel,
        out_type=jax.ShapeDtypeStruct((ep // CHUNK, CHUNK), jnp.float32),
        mesh=_sc_mesh(),
        compiler_params=pltpu.CompilerParams(use_tc_tiling_on_sc=False),
        scratch_types=[
            pltpu.VMEM((2, nsw, CHUNK), jnp.int32),
            pltpu.VMEM((2, nsw, CHUNK), jnp.int32),
            [pltpu.VMEM((nsw * CHUNK,), jnp.float32) for _ in range(6)],
            pltpu.VMEM((nsw, CHUNK), jnp.float32),
            pltpu.SemaphoreType.DMA,
            pltpu.SemaphoreType.DMA,
        ],
    )
    def k(sx_hbm, sy_hbm, sz_hbm, dx_hbm, dy_hbm, dz_hbm,
          src_hbm, dst_hbm, win_hbm,
          sidx_v, didx_v, crd_v, wout_v, semi, semg):
        c = lax.axis_index("c")
        s = lax.axis_index("s")
        brow = (c * NS + s) * nchunk
        iota = lax.iota(jnp.int32, LN)

        def idx_copies(b, p):
            r = brow + b * nsw
            return [
                pltpu.make_async_copy(
                    src_hbm.at[pl.ds(r, nsw)], sidx_v.at[p], semi),
                pltpu.make_async_copy(
                    dst_hbm.at[pl.ds(r, nsw)], didx_v.at[p], semi),
            ]

        def issue_idx(b, p):
            for cp in idx_copies(b, p):
                cp.start()

        def process(b, p):
            tbls = [sx_hbm, sy_hbm, sz_hbm, dx_hbm, dy_hbm, dz_hbm]
            gs = []
            for q in range(nsw):
                for t in range(6):
                    idx = sidx_v if t < 3 else didx_v
                    gs.append(pltpu.async_copy(
                        tbls[t].at[idx.at[p, q]],
                        crd_v[t].at[pl.ds(q * CHUNK, CHUNK)], semg))
            for q in range(nsw):
                for t in range(6):
                    gs[q * 6 + t].wait()

                def sub(i, _, q=q):
                    sl = pl.ds(q * CHUNK + i * LN, LN)
                    dx = crd_v[3][sl] - crd_v[0][sl]
                    dy = crd_v[4][sl] - crd_v[1][sl]
                    dz = crd_v[5][sl] - crd_v[2][sl]
                    r2 = (dx * dx + dy * dy + dz * dz) * INV_H2
                    t1 = 1.0 - r2
                    w = t1 * t1 * t1
                    w = jnp.minimum(jnp.maximum(w, 0.0), 1.0)
                    g = (brow + b * nsw + q) * CHUNK + i * LN + iota
                    w = jnp.where(g < e_real, w, 0.0)
                    wout_v[q, pl.ds(i * LN, LN)] = w
                    return 0

                lax.fori_loop(0, CHUNK // LN, sub, 0)
            pltpu.sync_copy(wout_v,
                            win_hbm.at[pl.ds(brow + b * nsw, nsw)])

        issue_idx(0, 0)

        def pair(i, _):
            b0 = 2 * i
            b1 = 2 * i + 1
            for cp in idx_copies(b0, 0):
                cp.wait()
            issue_idx(b1, 1)
            process(b0, 0)
            for cp in idx_copies(b1, 1):
                cp.wait()

            @pl.when(b1 + 1 < nsup)
            def _():
                issue_idx(b1 + 1, 0)
            process(b1, 1)
            return 0

        lax.fori_loop(0, nsup // 2, pair, 0)

    return k


NSUP = 4


def _seg_kernel(n, ep, width, colsplit):
    per_sc = ep if colsplit else ep // NC
    per_tile = per_sc // NS
    nchunk = per_tile // CHUNK
    nsup = nchunk // NSUP
    rpt = ((_cdiv(n, NS) + 7) // 8) * 8
    assert per_tile % (CHUNK * NSUP) == 0 and nsup % 2 == 0
    assert width % LN == 0 and (n - rpt) % 8 == 0

    @functools.partial(
        pl.kernel,
        out_type=jax.ShapeDtypeStruct((NC, n, width), jnp.float32),
        mesh=_sc_mesh(),
        compiler_params=pltpu.CompilerParams(use_tc_tiling_on_sc=False),
        scratch_types=[
            pltpu.VMEM((2, NSUP, CHUNK), jnp.int32),
            pltpu.VMEM((2, NSUP, CHUNK), jnp.int32),
            pltpu.VMEM((2, NSUP, CHUNK), jnp.float32),
            pltpu.VMEM((NSUP * CHUNK, width), jnp.float32),
            pltpu.VMEM_SHARED((n, width), jnp.float32),
            pltpu.SemaphoreType.DMA,
            pltpu.SemaphoreType.DMA,
            pltpu.SemaphoreType.DMA,
        ],
    )
    def k(tbl_hbm, src_hbm, dst_hbm, win_hbm, zeros_hbm, out_hbm,
          sidx_v, didx_v, win_v, rows_v, accum_sh, semi, semg, semsc):
        c = lax.axis_index("c")
        s = lax.axis_index("s")
        if colsplit:
            erow0 = s * nchunk
            srow0 = c * (ep // CHUNK) + s * nchunk
        else:
            erow0 = c * (per_sc // CHUNK) + s * nchunk
            srow0 = erow0
        row0 = pl.multiple_of(jnp.minimum(s * rpt, n - rpt), 8)

        pltpu.sync_copy(zeros_hbm.at[pl.ds(row0, rpt)],
                        accum_sh.at[pl.ds(row0, rpt)])
        plsc.subcore_barrier()

        def idx_copies(b, p):
            r = b * NSUP
            return [
                pltpu.make_async_copy(
                    src_hbm.at[pl.ds(srow0 + r, NSUP)], sidx_v.at[p], semi),
                pltpu.make_async_copy(
                    dst_hbm.at[pl.ds(erow0 + r, NSUP)], didx_v.at[p], semi),
                pltpu.make_async_copy(
                    win_hbm.at[pl.ds(erow0 + r, NSUP)], win_v.at[p], semi),
            ]

        def issue_idx(b, p):
            for cp in idx_copies(b, p):
                cp.start()

        def process(b, p):
            gs = []
            for q in range(NSUP):
                gs.append(pltpu.async_copy(
                    tbl_hbm.at[sidx_v.at[p, q]],
                    rows_v.at[pl.ds(q * CHUNK, CHUNK)], semg))
            ss = []
            for q in range(NSUP):
                gs[q].wait()

                def sgrp(g, _, q=q):
                    win16 = win_v[p, q, pl.ds(g * LN, LN)]
                    for lane in range(LN):
                        w = win16[lane]
                        i = q * CHUNK + g * LN + lane
                        for colb in range(width // LN):
                            sl = pl.ds(colb * LN, LN)
                            rows_v[i, sl] = rows_v[i, sl] * w
                    return 0
                lax.fori_loop(0, CHUNK // LN, sgrp, 0)
                ss.append(pltpu.async_copy(
                    rows_v.at[pl.ds(q * CHUNK, CHUNK)],
                    accum_sh.at[didx_v.at[p, q]], semsc, add=True))
            for cp in ss:
                cp.wait()

        issue_idx(0, 0)

        def pair(i, _):
            b0 = 2 * i
            b1 = 2 * i + 1
            for cp in idx_copies(b0, 0):
                cp.wait()
            issue_idx(b1, 1)
            process(b0, 0)
            for cp in idx_copies(b1, 1):
                cp.wait()

            @pl.when(b1 + 1 < nsup)
            def _():
                issue_idx(b1 + 1, 0)
            process(b1, 1)
            return 0

        lax.fori_loop(0, nsup // 2, pair, 0)
        plsc.subcore_barrier()
        pltpu.sync_copy(accum_sh.at[pl.ds(row0, rpt)],
                        out_hbm.at[c, pl.ds(row0, rpt)])

    return k


def kernel(pos, vel, box, box_feats, edge_index, box_src, box_dst,
           W0f, W0o, Wd0, bd0, Wc1, Wd1, bd1, Wc2, Wd2, bd2, Wc3, Wd3, bd3):
    n = pos.shape[0]
    nb = box.shape[0]
    e = edge_index.shape[1]
    eb = box_src.shape[0]
    grain = NC * NS * CHUNK * 2 * NSUP
    ep = _cdiv(e, grain) * grain
    ebp = _cdiv(eb, grain) * grain

    pad1 = ((0, 0), (0, 1))
    pos4 = jnp.pad(pos, pad1)
    vel4 = jnp.pad(vel, pad1)
    bf4 = jnp.pad(box_feats, pad1)
    w0o4 = jnp.pad(W0o, ((0, 1), (0, 0)))

    wfs = jnp.pad(W0f[1:4], ((0, 1), (0, 0)))
    bf0 = W0f[0:1]
    wds = jnp.pad(Wd0[1:4], ((0, 1), (0, 0)))
    bd0r = Wd0[0:1] + bd0[None, :]

    wc1a, wc1b = Wc1[:, :32], Wc1[:, 32:]
    wc2a, wc2b = Wc2[:, :32], Wc2[:, 32:]
    wc3p = jnp.pad(Wc3, ((0, 0), (0, 13)))
    wd3p = jnp.pad(Wd3, ((0, 0), (0, 13)))
    bd3p = jnp.pad(bd3, (0, 13))[None, :]

    srcp = jnp.pad(edge_index[0], (0, ep - e))
    dstp = jnp.pad(edge_index[1], (0, ep - e))
    bsrcp = jnp.pad(box_src, (0, ebp - eb))
    bdstp = jnp.pad(box_dst, (0, ebp - eb))
    src_off = jnp.concatenate([srcp, srcp + n])
    src2 = srcp.reshape(-1, CHUNK)
    dst2 = dstp.reshape(-1, CHUNK)
    bsrc2 = bsrcp.reshape(-1, CHUNK)
    bdst2 = bdstp.reshape(-1, CHUNK)
    srco2 = src_off.reshape(-1, CHUNK)

    zeros32 = jnp.zeros((n, 32), jnp.float32)
    zeros16 = jnp.zeros((n, 16), jnp.float32)

    pos2p, yf, d0 = _stage0(pos4, vel4, wfs, bf0, wds, bd0r)
    ybox = _boxmm(bf4, w0o4)

    p2x, p2y, p2z = pos2p[:, 0], pos2p[:, 1], pos2p[:, 2]
    bx, by, bz = box[:, 0], box[:, 1], box[:, 2]
    winf = _win_kernel(e, ep)(p2x, p2y, p2z, p2x, p2y, p2z, src2, dst2)
    winb = _win_kernel(eb, ebp)(bx, by, bz, p2x, p2y, p2z, bsrc2, bdst2)

    sf = _seg_kernel(n, ep, 32, False)(yf, src2, dst2, winf, zeros32)
    so = _seg_kernel(n, ebp, 32, False)(ybox, bsrc2, bdst2, winb, zeros32)

    y1a, y1b, d1 = _stage1(so, sf, d0, wc1a, wc1b, Wd1, bd1[None, :])
    y1t = jnp.concatenate([y1a, y1b], axis=0)
    s1 = _seg_kernel(n, ep, 32, True)(y1t, srco2, dst2, winf, zeros32)

    y2a, y2b, d2 = _stage2(s1, d1, wc2a, wc2b, Wd2, bd2[None, :])
    y2t = jnp.concatenate([y2a, y2b], axis=0)
    s2 = _seg_kernel(n, ep, 32, True)(y2t, srco2, dst2, winf, zeros32)

    y3, d3 = _stage3(s2, d2, wc3p, wd3p, bd3p)
    s3 = _seg_kernel(n, ep, 16, False)(y3, src2, dst2, winf, zeros16)

    out = (s3[0] + s3[1] + d3)[:, :3] * (1.0 / 128.0)
    return out


# speedup vs baseline: 7.4102x; 1.0198x over previous
import functools

import jax
import jax.numpy as jnp
from jax import lax
from jax.experimental import pallas as pl
from jax.experimental.pallas import tpu as pltpu
from jax.experimental.pallas import tpu_sc as plsc

NC = 2
NS = 16
LN = 16
CHUNK = 128

DT = 0.02
INV_H2 = float(1.0 / ((1.5 * 6 * 0.025 / 2.0) ** 2))


def _cdiv(a, b):
    return (a + b - 1) // b


def _row_spec(blk, w):
    return pl.BlockSpec((blk, w), lambda i: (i, 0))


def _full_spec(shape):
    return pl.BlockSpec(shape, lambda i: tuple(0 for _ in shape))


def _stage0_body(pos_ref, vel_ref, dtg_ref, wf_ref, bf_ref, wd_ref, bd_ref,
                 pos2_ref, yf_ref, d0_ref):
    vel = vel_ref[...]
    vel2 = vel + dtg_ref[...]
    pos2_ref[...] = pos_ref[...] + (0.5 * DT) * (vel2 + vel)
    yf_ref[...] = jnp.dot(vel2, wf_ref[...],
                          preferred_element_type=jnp.float32) + bf_ref[...]
    d0_ref[...] = jnp.dot(vel2, wd_ref[...],
                          preferred_element_type=jnp.float32) + bd_ref[...]


def _stage0(pos4, vel4, wfs, bf, wds, bd, blk=2000):
    n = pos4.shape[0]
    grid = n // blk
    dtg = jnp.array([[0.0, -9.81 * DT, 0.0, 0.0]], dtype=jnp.float32)
    return pl.pallas_call(
        _stage0_body,
        grid=(grid,),
        in_specs=[_row_spec(blk, 4), _row_spec(blk, 4), _full_spec((1, 4)),
                  _full_spec((4, 32)), _full_spec((1, 32)),
                  _full_spec((4, 32)), _full_spec((1, 32))],
        out_specs=[_row_spec(blk, 4), _row_spec(blk, 32), _row_spec(blk, 32)],
        out_shape=[jax.ShapeDtypeStruct((n, 4), jnp.float32),
                   jax.ShapeDtypeStruct((n, 32), jnp.float32),
                   jax.ShapeDtypeStruct((n, 32), jnp.float32)],
    )(pos4, vel4, dtg, wfs, bf, wds, bd)


def _boxmm_body(bf_ref, w_ref, y_ref):
    y_ref[...] = jnp.dot(bf_ref[...], w_ref[...],
                         preferred_element_type=jnp.float32)


def _boxmm(bf4, w0o4, blk=2000):
    nb = bf4.shape[0]
    return pl.pallas_call(
        _boxmm_body,
        grid=(nb // blk,),
        in_specs=[_row_spec(blk, 4), _full_spec((4, 32))],
        out_specs=_row_spec(blk, 32),
        out_shape=jax.ShapeDtypeStruct((nb, 32), jnp.float32),
    )(bf4, w0o4)


def _stage1_body(so_ref, sf_ref, d0_ref, wca_ref, wcb_ref, wd_ref, bd_ref,
                 ya_ref, yb_ref, d1_ref):
    h = jnp.concatenate(
        [so_ref[0] + so_ref[1], sf_ref[0] + sf_ref[1], d0_ref[...]], axis=-1)
    h = jnp.maximum(h, 0.0)
    ya_ref[...] = jnp.dot(h, wca_ref[...], preferred_element_type=jnp.float32)
    yb_ref[...] = jnp.dot(h, wcb_ref[...], preferred_element_type=jnp.float32)
    d1_ref[...] = jnp.dot(h, wd_ref[...],
                          preferred_element_type=jnp.float32) + bd_ref[...]


def _stage1(so, sf, d0, wca, wcb, wd, bd, blk=2000):
    n = d0.shape[0]
    sspec = pl.BlockSpec((2, blk, 32), lambda i: (0, i, 0))
    return pl.pallas_call(
        _stage1_body,
        grid=(n // blk,),
        in_specs=[sspec, sspec, _row_spec(blk, 32),
                  _full_spec((96, 32)), _full_spec((96, 32)),
                  _full_spec((96, 64)), _full_spec((1, 64))],
        out_specs=[_row_spec(blk, 32), _row_spec(blk, 32), _row_spec(blk, 64)],
        out_shape=[jax.ShapeDtypeStruct((n, 32), jnp.float32),
                   jax.ShapeDtypeStruct((n, 32), jnp.float32),
                   jax.ShapeDtypeStruct((n, 64), jnp.float32)],
    )(so, sf, d0, wca, wcb, wd, bd)


def _stage2_body(s_ref, d_ref, wca_ref, wcb_ref, wd_ref, bd_ref,
                 ya_ref, yb_ref, d2_ref):
    ans = jnp.concatenate([s_ref[0], s_ref[1]], axis=-1) + d_ref[...]
    h = jnp.maximum(ans, 0.0)
    ya_ref[...] = jnp.dot(h, wca_ref[...], preferred_element_type=jnp.float32)
    yb_ref[...] = jnp.dot(h, wcb_ref[...], preferred_element_type=jnp.float32)
    d2_ref[...] = jnp.dot(h, wd_ref[...],
                          preferred_element_type=jnp.float32) + bd_ref[...] + ans


def _stage2(s1, d1, wca, wcb, wd, bd, blk=2000):
    n = d1.shape[0]
    sspec = pl.BlockSpec((2, blk, 32), lambda i: (0, i, 0))
    return pl.pallas_call(
        _stage2_body,
        grid=(n // blk,),
        in_specs=[sspec, _row_spec(blk, 64),
                  _full_spec((64, 32)), _full_spec((64, 32)),
                  _full_spec((64, 64)), _full_spec((1, 64))],
        out_specs=[_row_spec(blk, 32), _row_spec(blk, 32), _row_spec(blk, 64)],
        out_shape=[jax.ShapeDtypeStruct((n, 32), jnp.float32),
                   jax.ShapeDtypeStruct((n, 32), jnp.float32),
                   jax.ShapeDtypeStruct((n, 64), jnp.float32)],
    )(s1, d1, wca, wcb, wd, bd)


def _stage3_body(s_ref, d_ref, wc_ref, wd_ref, bd_ref, y3_ref, d3_ref):
    ans = jnp.concatenate([s_ref[0], s_ref[1]], axis=-1) + d_ref[...]
    h = jnp.maximum(ans, 0.0)
    y3_ref[...] = jnp.dot(h, wc_ref[...], preferred_element_type=jnp.float32)
    d3_ref[...] = jnp.dot(h, wd_ref[...],
                          preferred_element_type=jnp.float32) + bd_ref[...]


def _stage3(s2, d2, wc3p, wd3p, bd3p, blk=2000):
    n = d2.shape[0]
    sspec = pl.BlockSpec((2, blk, 32), lambda i: (0, i, 0))
    return pl.pallas_call(
        _stage3_body,
        grid=(n // blk,),
        in_specs=[sspec, _row_spec(blk, 64),
                  _full_spec((64, 16)), _full_spec((64, 16)),
                  _full_spec((1, 16))],
        out_specs=[_row_spec(blk, 16), _row_spec(blk, 16)],
        out_shape=[jax.ShapeDtypeStruct((n, 16), jnp.float32),
                   jax.ShapeDtypeStruct((n, 16), jnp.float32)],
    )(s2, d2, wc3p, wd3p, bd3p)


def _sc_mesh():
    return plsc.VectorSubcoreMesh(core_axis_name="c", subcore_axis_name="s")


def _win_kernel(e_real, ep):
    per_tile = ep // (NC * NS)
    nchunk = per_tile // CHUNK
    nsw = NSUP
    nsup = nchunk // nsw
    assert nchunk % (2 * nsw) == 0

    @functools.partial(
        pl.kernel,
        out_type=jax.ShapeDtypeStruct((ep // CHUNK, CHUNK), jnp.float32),
        mesh=_sc_mesh(),
        compiler_params=pltpu.CompilerParams(use_tc_tiling_on_sc=False),
        scratch_types=[
            pltpu.VMEM((2, nsw, CHUNK), jnp.int32),
            pltpu.VMEM((2, nsw, CHUNK), jnp.int32),
            [pltpu.VMEM((nsw * CHUNK,), jnp.float32) for _ in range(6)],
            pltpu.VMEM((nsw, CHUNK), jnp.float32),
            pltpu.SemaphoreType.DMA,
            pltpu.SemaphoreType.DMA,
        ],
    )
    def k(sx_hbm, sy_hbm, sz_hbm, dx_hbm, dy_hbm, dz_hbm,
          src_hbm, dst_hbm, win_hbm,
          sidx_v, didx_v, crd_v, wout_v, semi, semg):
        c = lax.axis_index("c")
        s = lax.axis_index("s")
        brow = (c * NS + s) * nchunk
        iota = lax.iota(jnp.int32, LN)

        def idx_copies(b, p):
            r = brow + b * nsw
            return [
                pltpu.make_async_copy(
                    src_hbm.at[pl.ds(r, nsw)], sidx_v.at[p], semi),
                pltpu.make_async_copy(
                    dst_hbm.at[pl.ds(r, nsw)], didx_v.at[p], semi),
            ]

        def issue_idx(b, p):
            for cp in idx_copies(b, p):
                cp.start()

        def process(b, p):
            tbls = [sx_hbm, sy_hbm, sz_hbm, dx_hbm, dy_hbm, dz_hbm]
            gs = []
            for q in range(nsw):
                for t in range(6):
                    idx = sidx_v if t < 3 else didx_v
                    gs.append(pltpu.async_copy(
                        tbls[t].at[idx.at[p, q]],
                        crd_v[t].at[pl.ds(q * CHUNK, CHUNK)], semg))
            for q in range(nsw):
                for t in range(6):
                    gs[q * 6 + t].wait()

                def sub(i, _, q=q):
                    sl = pl.ds(q * CHUNK + i * LN, LN)
                    dx = crd_v[3][sl] - crd_v[0][sl]
                    dy = crd_v[4][sl] - crd_v[1][sl]
                    dz = crd_v[5][sl] - crd_v[2][sl]
                    r2 = (dx * dx + dy * dy + dz * dz) * INV_H2
                    t1 = 1.0 - r2
                    w = t1 * t1 * t1
                    w = jnp.minimum(jnp.maximum(w, 0.0), 1.0)
                    g = (brow + b * nsw + q) * CHUNK + i * LN + iota
                    w = jnp.where(g < e_real, w, 0.0)
                    wout_v[q, pl.ds(i * LN, LN)] = w
                    return 0

                lax.fori_loop(0, CHUNK // LN, sub, 0)
            pltpu.sync_copy(wout_v,
                            win_hbm.at[pl.ds(brow + b * nsw, nsw)])

        issue_idx(0, 0)

        def pair(i, _):
            b0 = 2 * i
            b1 = 2 * i + 1
            for cp in idx_copies(b0, 0):
                cp.wait()
            issue_idx(b1, 1)
            process(b0, 0)
            for cp in idx_copies(b1, 1):
                cp.wait()

            @pl.when(b1 + 1 < nsup)
            def _():
                issue_idx(b1 + 1, 0)
            process(b1, 1)
            return 0

        lax.fori_loop(0, nsup // 2, pair, 0)

    return k


NSUP = 4


def _seg_kernel(n, ep, width, colsplit):
    per_sc = ep if colsplit else ep // NC
    per_tile = per_sc // NS
    nchunk = per_tile // CHUNK
    nsup = nchunk // NSUP
    rpt = ((_cdiv(n, NS) + 7) // 8) * 8
    assert per_tile % (CHUNK * NSUP) == 0 and nsup % 2 == 0
    assert width % LN == 0 and (n - rpt) % 8 == 0

    @functools.partial(
        pl.kernel,
        out_type=jax.ShapeDtypeStruct((NC, n, width), jnp.float32),
        mesh=_sc_mesh(),
        compiler_params=pltpu.CompilerParams(use_tc_tiling_on_sc=False),
        scratch_types=[
            pltpu.VMEM((2, NSUP, CHUNK), jnp.int32),
            pltpu.VMEM((2, NSUP, CHUNK), jnp.int32),
            pltpu.VMEM((2, NSUP, CHUNK), jnp.float32),
            pltpu.VMEM((NSUP * CHUNK, width), jnp.float32),
            pltpu.VMEM_SHARED((n, width), jnp.float32),
            pltpu.SemaphoreType.DMA,
            pltpu.SemaphoreType.DMA,
            pltpu.SemaphoreType.DMA,
        ],
    )
    def k(tbl_hbm, src_hbm, dst_hbm, win_hbm, zeros_hbm, out_hbm,
          sidx_v, didx_v, win_v, rows_v, accum_sh, semi, semg, semsc):
        c = lax.axis_index("c")
        s = lax.axis_index("s")
        if colsplit:
            erow0 = s * nchunk
            srow0 = c * (ep // CHUNK) + s * nchunk
        else:
            erow0 = c * (per_sc // CHUNK) + s * nchunk
            srow0 = erow0
        row0 = pl.multiple_of(jnp.minimum(s * rpt, n - rpt), 8)

        pltpu.sync_copy(zeros_hbm.at[pl.ds(row0, rpt)],
                        accum_sh.at[pl.ds(row0, rpt)])
        plsc.subcore_barrier()

        def idx_copies(b, p):
            r = b * NSUP
            return [
                pltpu.make_async_copy(
                    src_hbm.at[pl.ds(srow0 + r, NSUP)], sidx_v.at[p], semi),
                pltpu.make_async_copy(
                    dst_hbm.at[pl.ds(erow0 + r, NSUP)], didx_v.at[p], semi),
                pltpu.make_async_copy(
                    win_hbm.at[pl.ds(erow0 + r, NSUP)], win_v.at[p], semi),
            ]

        def issue_idx(b, p):
            for cp in idx_copies(b, p):
                cp.start()

        def process(b, p):
            gs = []
            for q in range(NSUP):
                @pl.when(b > 0)
                def _(q=q):
                    pltpu.make_async_copy(
                        rows_v.at[pl.ds(q * CHUNK, CHUNK)],
                        accum_sh.at[didx_v.at[1 - p, q]], semsc).wait()
                gs.append(pltpu.async_copy(
                    tbl_hbm.at[sidx_v.at[p, q]],
                    rows_v.at[pl.ds(q * CHUNK, CHUNK)], semg))
            for q in range(NSUP):
                gs[q].wait()

                def sgrp(g, _, q=q):
                    win16 = win_v[p, q, pl.ds(g * LN, LN)]
                    for lane in range(LN):
                        w = win16[lane]
                        i = q * CHUNK + g * LN + lane
                        for colb in range(width // LN):
                            sl = pl.ds(colb * LN, LN)
                            rows_v[i, sl] = rows_v[i, sl] * w
                    return 0
                lax.fori_loop(0, CHUNK // LN, sgrp, 0)
                pltpu.async_copy(
                    rows_v.at[pl.ds(q * CHUNK, CHUNK)],
                    accum_sh.at[didx_v.at[p, q]], semsc, add=True)

        issue_idx(0, 0)

        def pair(i, _):
            b0 = 2 * i
            b1 = 2 * i + 1
            for cp in idx_copies(b0, 0):
                cp.wait()
            issue_idx(b1, 1)
            process(b0, 0)
            for cp in idx_copies(b1, 1):
                cp.wait()

            @pl.when(b1 + 1 < nsup)
            def _():
                issue_idx(b1 + 1, 0)
            process(b1, 1)
            return 0

        lax.fori_loop(0, nsup // 2, pair, 0)
        for q in range(NSUP):
            pltpu.make_async_copy(
                rows_v.at[pl.ds(q * CHUNK, CHUNK)],
                accum_sh.at[didx_v.at[1, q]], semsc).wait()
        plsc.subcore_barrier()
        pltpu.sync_copy(accum_sh.at[pl.ds(row0, rpt)],
                        out_hbm.at[c, pl.ds(row0, rpt)])

    return k


def kernel(pos, vel, box, box_feats, edge_index, box_src, box_dst,
           W0f, W0o, Wd0, bd0, Wc1, Wd1, bd1, Wc2, Wd2, bd2, Wc3, Wd3, bd3):
    n = pos.shape[0]
    nb = box.shape[0]
    e = edge_index.shape[1]
    eb = box_src.shape[0]
    grain = NC * NS * CHUNK * 2 * NSUP
    ep = _cdiv(e, grain) * grain
    ebp = _cdiv(eb, grain) * grain

    pad1 = ((0, 0), (0, 1))
    pos4 = jnp.pad(pos, pad1)
    vel4 = jnp.pad(vel, pad1)
    bf4 = jnp.pad(box_feats, pad1)
    w0o4 = jnp.pad(W0o, ((0, 1), (0, 0)))

    wfs = jnp.pad(W0f[1:4], ((0, 1), (0, 0)))
    bf0 = W0f[0:1]
    wds = jnp.pad(Wd0[1:4], ((0, 1), (0, 0)))
    bd0r = Wd0[0:1] + bd0[None, :]

    wc1a, wc1b = Wc1[:, :32], Wc1[:, 32:]
    wc2a, wc2b = Wc2[:, :32], Wc2[:, 32:]
    wc3p = jnp.pad(Wc3, ((0, 0), (0, 13)))
    wd3p = jnp.pad(Wd3, ((0, 0), (0, 13)))
    bd3p = jnp.pad(bd3, (0, 13))[None, :]

    srcp = jnp.pad(edge_index[0], (0, ep - e))
    dstp = jnp.pad(edge_index[1], (0, ep - e))
    bsrcp = jnp.pad(box_src, (0, ebp - eb))
    bdstp = jnp.pad(box_dst, (0, ebp - eb))
    src_off = jnp.concatenate([srcp, srcp + n])
    src2 = srcp.reshape(-1, CHUNK)
    dst2 = dstp.reshape(-1, CHUNK)
    bsrc2 = bsrcp.reshape(-1, CHUNK)
    bdst2 = bdstp.reshape(-1, CHUNK)
    srco2 = src_off.reshape(-1, CHUNK)

    zeros32 = jnp.zeros((n, 32), jnp.float32)
    zeros16 = jnp.zeros((n, 16), jnp.float32)

    pos2p, yf, d0 = _stage0(pos4, vel4, wfs, bf0, wds, bd0r)
    ybox = _boxmm(bf4, w0o4)

    p2x, p2y, p2z = pos2p[:, 0], pos2p[:, 1], pos2p[:, 2]
    bx, by, bz = box[:, 0], box[:, 1], box[:, 2]
    winf = _win_kernel(e, ep)(p2x, p2y, p2z, p2x, p2y, p2z, src2, dst2)
    winb = _win_kernel(eb, ebp)(bx, by, bz, p2x, p2y, p2z, bsrc2, bdst2)

    sf = _seg_kernel(n, ep, 32, False)(yf, src2, dst2, winf, zeros32)
    so = _seg_kernel(n, ebp, 32, False)(ybox, bsrc2, bdst2, winb, zeros32)

    y1a, y1b, d1 = _stage1(so, sf, d0, wc1a, wc1b, Wd1, bd1[None, :])
    y1t = jnp.concatenate([y1a, y1b], axis=0)
    s1 = _seg_kernel(n, ep, 32, True)(y1t, srco2, dst2, winf, zeros32)

    y2a, y2b, d2 = _stage2(s1, d1, wc2a, wc2b, Wd2, bd2[None, :])
    y2t = jnp.concatenate([y2a, y2b], axis=0)
    s2 = _seg_kernel(n, ep, 32, True)(y2t, srco2, dst2, winf, zeros32)

    y3, d3 = _stage3(s2, d2, wc3p, wd3p, bd3p)
    s3 = _seg_kernel(n, ep, 16, False)(y3, src2, dst2, winf, zeros16)

    out = (s3[0] + s3[1] + d3)[:, :3] * (1.0 / 128.0)
    return out
